# Initial kernel scaffold; baseline (speedup 1.0000x reference)
import jax
import jax.numpy as jnp
from jax.experimental import pallas as pl


def kernel(images, patch_proj, embed_table, end_token_ids):
    raise NotImplementedError("write your pallas kernel here")


# speedup vs baseline: 2.1871x; 2.1871x over previous
import functools

import jax
import jax.numpy as jnp
from jax import lax
from jax.experimental import pallas as pl
from jax.experimental.pallas import tpu as pltpu
from jax.experimental.pallas import tpu_sc as plsc

_B, _C, _H, _W = 8, 3, 384, 384
_P = 16
_HP = _H // _P
_WP = _W // _P
_NP = _HP * _WP
_DP = _C * _P * _P
_DM = 2048
_VOCAB = 32000


def _sc_gather_body(table_hbm, idx_hbm, out_hbm, idx_v, row_v, sem):
    cid = lax.axis_index("c")
    sid = lax.axis_index("s")
    wid = sid * 2 + cid

    @pl.when(wid == 0)
    def _():
        pltpu.sync_copy(idx_hbm, idx_v)
        pltpu.async_copy(table_hbm.at[idx_v], row_v, sem).wait()
        pltpu.sync_copy(row_v, out_hbm)


def _sc_gather(embed_table, end_token_ids):
    mesh = plsc.VectorSubcoreMesh(core_axis_name="c", subcore_axis_name="s")
    k = functools.partial(
        pl.kernel,
        mesh=mesh,
        out_type=jax.ShapeDtypeStruct((1, _DM), jnp.float32),
        scratch_types=[
            pltpu.VMEM((1,), jnp.int32),
            pltpu.VMEM((1, _DM), jnp.float32),
            pltpu.SemaphoreType.DMA,
        ],
    )(_sc_gather_body)
    return k(embed_table, end_token_ids)


def _tc_body(img_ref, w_ref, end_ref, out_ref):
    x = img_ref[0]
    x = x.reshape(_C, _HP, _P, _WP, _P)
    x = x.transpose(1, 3, 0, 2, 4).reshape(_NP, _DP)
    out_ref[0, :_NP, :] = jnp.dot(x, w_ref[...], preferred_element_type=jnp.float32)
    out_ref[0, _NP:, :] = end_ref[...]


def kernel(images, patch_proj, embed_table, end_token_ids):
    end_row = _sc_gather(embed_table, end_token_ids)
    out = pl.pallas_call(
        _tc_body,
        grid=(_B,),
        in_specs=[
            pl.BlockSpec((1, _C, _H, _W), lambda b: (b, 0, 0, 0)),
            pl.BlockSpec((_DP, _DM), lambda b: (0, 0)),
            pl.BlockSpec((1, _DM), lambda b: (0, 0)),
        ],
        out_specs=pl.BlockSpec((1, _NP + 1, _DM), lambda b: (b, 0, 0)),
        out_shape=jax.ShapeDtypeStruct((_B, _NP + 1, _DM), jnp.float32),
    )(images, patch_proj, end_row)
    return out


# speedup vs baseline: 2.9851x; 1.3648x over previous
import functools

import jax
import jax.numpy as jnp
from jax import lax
from jax.experimental import pallas as pl
from jax.experimental.pallas import tpu as pltpu
from jax.experimental.pallas import tpu_sc as plsc

_B, _C, _H, _W = 8, 3, 384, 384
_P = 16
_HP = _H // _P
_WP = _W // _P
_NP = _HP * _WP
_DP = _C * _P * _P
_DM = 2048
_VOCAB = 32000


def _sc_gather_body(table_hbm, idx_hbm, out_hbm, idx_v, row_v, sem):
    cid = lax.axis_index("c")
    sid = lax.axis_index("s")
    wid = sid * 2 + cid

    @pl.when(wid == 0)
    def _():
        pltpu.sync_copy(idx_hbm, idx_v)
        pltpu.async_copy(table_hbm.at[idx_v], row_v, sem).wait()
        pltpu.sync_copy(row_v, out_hbm)


def _sc_gather(embed_table, end_token_ids):
    mesh = plsc.VectorSubcoreMesh(core_axis_name="c", subcore_axis_name="s")
    k = functools.partial(
        pl.kernel,
        mesh=mesh,
        out_type=jax.ShapeDtypeStruct((1, _DM), jnp.float32),
        scratch_types=[
            pltpu.VMEM((1,), jnp.int32),
            pltpu.VMEM((1, _DM), jnp.float32),
            pltpu.SemaphoreType.DMA,
        ],
    )(_sc_gather_body)
    return k(embed_table, end_token_ids)


def _tc_body(img_ref, w_ref, end_ref, out_ref):
    i = pl.program_id(0)

    @pl.when(i < _HP)
    def _():
        x = img_ref[...]
        x = x.reshape(_B, _C, _P, _WP, _P)
        x = x.transpose(3, 0, 1, 2, 4)
        a = x.reshape(_WP * _B, _DP)
        f = jnp.dot(a, w_ref[...], preferred_element_type=jnp.float32)
        out_ref[...] = f.reshape(_WP, _B, _DM)

    @pl.when(i == _HP)
    def _():
        out_ref[...] = jnp.broadcast_to(end_ref[...][None], (_WP, _B, _DM))


def kernel(images, patch_proj, embed_table, end_token_ids):
    end_row = _sc_gather(embed_table, end_token_ids)
    out = pl.pallas_call(
        _tc_body,
        grid=(_HP + 1,),
        in_specs=[
            pl.BlockSpec((_B, _C, _P, _W), lambda i: (0, 0, jnp.minimum(i, _HP - 1), 0)),
            pl.BlockSpec((_DP, _DM), lambda i: (0, 0)),
            pl.BlockSpec((1, _DM), lambda i: (0, 0)),
        ],
        out_specs=pl.BlockSpec((_WP, _B, _DM), lambda i: (i, 0, 0)),
        out_shape=jax.ShapeDtypeStruct((_NP + 1, _B, _DM), jnp.float32),
    )(images, patch_proj, end_row)
    return out.transpose(1, 0, 2)


# speedup vs baseline: 3.6874x; 1.2353x over previous
import functools

import jax
import jax.numpy as jnp
from jax import lax
from jax.experimental import pallas as pl
from jax.experimental.pallas import tpu as pltpu
from jax.experimental.pallas import tpu_sc as plsc

_B, _C, _H, _W = 8, 3, 384, 384
_P = 16
_HP = _H // _P
_WP = _W // _P
_NP = _HP * _WP
_DP = _C * _P * _P
_DM = 2048
_VOCAB = 32000


def _sc_gather_body(table_hbm, idx_hbm, out_hbm, idx_v, row_v, sem):
    cid = lax.axis_index("c")
    sid = lax.axis_index("s")
    wid = sid * 2 + cid

    @pl.when(wid == 0)
    def _():
        pltpu.sync_copy(idx_hbm, idx_v)
        pltpu.async_copy(table_hbm.at[idx_v], row_v, sem).wait()
        pltpu.sync_copy(row_v, out_hbm)


def _sc_gather(embed_table, end_token_ids):
    mesh = plsc.VectorSubcoreMesh(core_axis_name="c", subcore_axis_name="s")
    k = functools.partial(
        pl.kernel,
        mesh=mesh,
        out_type=jax.ShapeDtypeStruct((1, _DM), jnp.float32),
        scratch_types=[
            pltpu.VMEM((1,), jnp.int32),
            pltpu.VMEM((1, _DM), jnp.float32),
            pltpu.SemaphoreType.DMA,
        ],
    )(_sc_gather_body)
    return k(embed_table, end_token_ids)


def _tc_body(img_ref, w_ref, end_ref, out_ref, wbf_ref):
    i = pl.program_id(0)

    @pl.when(i == 0)
    def _():
        wbf_ref[...] = w_ref[...].astype(jnp.bfloat16)

    @pl.when(i < _HP)
    def _():
        x = img_ref[...].astype(jnp.bfloat16)
        x = x.reshape(_B, _C, _P, _WP, _P)
        x = x.transpose(3, 0, 1, 2, 4)
        a = x.reshape(_WP * _B, _DP)
        f = jnp.dot(a, wbf_ref[...], preferred_element_type=jnp.float32)
        out_ref[...] = f.reshape(_WP, _B, _DM)

    @pl.when(i == _HP)
    def _():
        out_ref[...] = jnp.broadcast_to(end_ref[...][None], (_WP, _B, _DM))


def kernel(images, patch_proj, embed_table, end_token_ids):
    end_row = _sc_gather(embed_table, end_token_ids)
    out = pl.pallas_call(
        _tc_body,
        grid=(_HP + 1,),
        in_specs=[
            pl.BlockSpec((_B, _C, _P, _W), lambda i: (0, 0, jnp.minimum(i, _HP - 1), 0)),
            pl.BlockSpec((_DP, _DM), lambda i: (0, 0)),
            pl.BlockSpec((1, _DM), lambda i: (0, 0)),
        ],
        out_specs=pl.BlockSpec((_WP, _B, _DM), lambda i: (i, 0, 0)),
        out_shape=jax.ShapeDtypeStruct((_NP + 1, _B, _DM), jnp.float32),
        scratch_shapes=[pltpu.VMEM((_DP, _DM), jnp.bfloat16)],
    )(images, patch_proj, end_row)
    return out.transpose(1, 0, 2)
